# Initial kernel scaffold; baseline (speedup 1.0000x reference)
import jax
import jax.numpy as jnp
from jax.experimental import pallas as pl


def kernel(nodes, arcs, edge_index, node_types, set_mask, output_mask, W_state_0, b_state_0, W_state_1, b_state_1, W_out_0, b_out_0, W_out_1, b_out_1):
    raise NotImplementedError("write your pallas kernel here")


# speedup vs baseline: 4.7963x; 4.7963x over previous
import functools

import jax
import jax.numpy as jnp
from jax import lax
from jax.experimental import pallas as pl
from jax.experimental.pallas import tpu as pltpu
from jax.experimental.pallas import tpu_sc as plsc

_N = 10000
_E = 320000
_D = 128
_NC, _NS = 2, 16
_NW = _NC * _NS
_RPT = 632
_NP = _NS * _RPT
_CH = 128
_NCHUNK = 79
_EPAD = _NW * _NCHUNK * _CH
_AW = 16
_BN = 1000
_GRID = _N // _BN

@functools.lru_cache(maxsize=None)
def _build_mp():
    mesh = plsc.VectorSubcoreMesh(
        core_axis_name="c", subcore_axis_name="s",
        num_cores=_NC, num_subcores=_NS)

    @functools.partial(
        pl.kernel,
        out_type=jax.ShapeDtypeStruct((_NC, _NP, _D), jnp.float32),
        mesh=mesh,
        scratch_types=[
            pltpu.VMEM((_NCHUNK, _CH), jnp.int32),
            pltpu.VMEM((_NCHUNK, _CH), jnp.int32),
            pltpu.VMEM((_CH, _D), jnp.float32),
            pltpu.VMEM_SHARED((_NP, _D), jnp.float32),
            pltpu.SemaphoreType.DMA,
        ],
    )
    def mp(state_hbm, src_hbm, dst_hbm, zeros_hbm, out_hbm,
           src_v, dst_v, rows_v, agg_sh, sem):
        c = lax.axis_index("c")
        s = lax.axis_index("s")
        wid = c * _NS + s
        base = s * _RPT
        pltpu.sync_copy(zeros_hbm.at[pl.ds(base, _RPT)],
                        agg_sh.at[pl.ds(base, _RPT)])
        pltpu.sync_copy(src_hbm.at[wid], src_v)
        pltpu.sync_copy(dst_hbm.at[wid], dst_v)
        plsc.subcore_barrier()

        @pl.loop(0, _NCHUNK)
        def _(j):
            pltpu.async_copy(state_hbm.at[src_v.at[j]], rows_v, sem).wait()
            pltpu.sync_copy(rows_v, agg_sh.at[dst_v.at[j]], add=True)

        plsc.subcore_barrier()
        pltpu.sync_copy(agg_sh.at[pl.ds(base, _RPT)],
                        out_hbm.at[c, pl.ds(base, _RPT)])

    return mp


@functools.lru_cache(maxsize=None)
def _build_arcs_agg():
    mesh = plsc.VectorSubcoreMesh(
        core_axis_name="c", subcore_axis_name="s",
        num_cores=_NC, num_subcores=_NS)

    @functools.partial(
        pl.kernel,
        out_type=jax.ShapeDtypeStruct((_NC, _NP, _D), jnp.float32),
        mesh=mesh,
        scratch_types=[
            pltpu.VMEM((_NCHUNK, _CH), jnp.int32),
            pltpu.VMEM((_CH, _AW), jnp.float32),
            pltpu.VMEM((_CH, _D), jnp.float32),
            pltpu.VMEM_SHARED((_NP, _D), jnp.float32),
        ],
    )
    def arcs_agg(arcs_hbm, dst_hbm, zeros_hbm, out_hbm,
                 dst_v, a16_v, rows_v, agg_sh):
        c = lax.axis_index("c")
        s = lax.axis_index("s")
        wid = c * _NS + s
        base = s * _RPT
        pltpu.sync_copy(zeros_hbm.at[pl.ds(base, _RPT)],
                        agg_sh.at[pl.ds(base, _RPT)])
        pltpu.sync_copy(dst_hbm.at[wid], dst_v)
        pltpu.sync_copy(zeros_hbm.at[pl.ds(0, _CH)], rows_v)
        plsc.subcore_barrier()

        @pl.loop(0, _NCHUNK)
        def _(j):
            pltpu.sync_copy(arcs_hbm.at[wid, j], a16_v)

            @pl.loop(0, _CH, unroll=8)
            def _(r):
                rows_v[r, pl.ds(0, _AW)] = a16_v[r, :]

            pltpu.sync_copy(rows_v, agg_sh.at[dst_v.at[j]], add=True)

        plsc.subcore_barrier()
        pltpu.sync_copy(agg_sh.at[pl.ds(base, _RPT)],
                        out_hbm.at[c, pl.ds(base, _RPT)])

    return arcs_agg


def _mp(*args):
    return _build_mp()(*args)


def _arcs_agg(*args):
    return _build_arcs_agg()(*args)


def _row_spec(d):
    return pl.BlockSpec((_BN, d), lambda i: (i, 0))


def _part_spec(d):
    return pl.BlockSpec((_NC, _BN, d), lambda i: (0, i, 0))


def _full_spec(shape):
    nd = len(shape)
    return pl.BlockSpec(shape, lambda i, _nd=nd: (0,) * _nd)


def _tc_call(body, in_specs, out_specs, out_shape):
    return pl.pallas_call(
        body, grid=(_GRID,), in_specs=in_specs, out_specs=out_specs,
        out_shape=out_shape,
        compiler_params=pltpu.CompilerParams(
            dimension_semantics=("parallel",)),
    )


def _pre0_body(nodes, pa, wl, we, b, t, l0, l1, st):
    aa = pa[0] + pa[1]
    x0 = (jnp.dot(nodes[...], wl[0], preferred_element_type=jnp.float32)
          + jnp.dot(aa, we[0], preferred_element_type=jnp.float32) + b[0])
    x1 = (jnp.dot(nodes[...], wl[1], preferred_element_type=jnp.float32)
          + jnp.dot(aa, we[1], preferred_element_type=jnp.float32) + b[1])
    l0[...] = x0
    l1[...] = x1
    st[...] = jnp.where(t[...] == 0, jnp.tanh(x0), jnp.tanh(x1))


def _iter_body(p, l0, l1, wa, t, st):
    agg = p[0] + p[1]
    s0 = jnp.tanh(jnp.dot(agg, wa[0], preferred_element_type=jnp.float32)
                  + l0[...])
    s1 = jnp.tanh(jnp.dot(agg, wa[1], preferred_element_type=jnp.float32)
                  + l1[...])
    st[...] = jnp.where(t[...] == 0, s0, s1)


def _out0_body(nodes, st, wn, ws, b, m, o):
    val = (jnp.dot(nodes[...], wn[...], preferred_element_type=jnp.float32)
           + jnp.dot(st[...], ws[...], preferred_element_type=jnp.float32)
           + b[...])
    o[...] = jnp.where(m[...] != 0, val, 0.0)


def _pre1_body(s0, o0, nodes, pa, wa, wb, wc, we, b, t, l0, l1, st):
    aa = pa[0] + pa[1]

    def term(i):
        return (jnp.dot(s0[...], wa[i], preferred_element_type=jnp.float32)
                + jnp.dot(o0[...], wb[i], preferred_element_type=jnp.float32)
                + jnp.dot(nodes[...], wc[i], preferred_element_type=jnp.float32)
                + jnp.dot(aa, we[i], preferred_element_type=jnp.float32)
                + b[i])

    x0 = term(0)
    x1 = term(1)
    l0[...] = x0
    l1[...] = x1
    st[...] = jnp.where(t[...] == 0, jnp.tanh(x0), jnp.tanh(x1))


def _out1_body(s0, o0, nodes, s1, ua, ub, uc, us, b, o):
    o[...] = (jnp.dot(s0[...], ua[...], preferred_element_type=jnp.float32)
              + jnp.dot(o0[...], ub[...], preferred_element_type=jnp.float32)
              + jnp.dot(nodes[...], uc[...], preferred_element_type=jnp.float32)
              + jnp.dot(s1[...], us[...], preferred_element_type=jnp.float32)
              + b[...])


def _f32(shape):
    return jax.ShapeDtypeStruct(shape, jnp.float32)


def kernel(nodes, arcs, edge_index, node_types, set_mask, output_mask,
           W_state_0, b_state_0, W_state_1, b_state_1,
           W_out_0, b_out_0, W_out_1, b_out_1):
    pad = _EPAD - _E
    dst = edge_index[0]
    src = edge_index[1]
    src_p = jnp.concatenate([src, jnp.zeros((pad,), jnp.int32)])
    src_p = src_p.reshape(_NW, _NCHUNK, _CH)
    dst_p = jnp.concatenate([dst, jnp.full((pad,), _N, jnp.int32)])
    dst_p = dst_p.reshape(_NW, _NCHUNK, _CH)
    arcs_p = jnp.zeros((_EPAD, _AW), jnp.float32).at[:_E, :4].set(arcs)
    arcs_p = arcs_p.reshape(_NW, _NCHUNK, _CH, _AW)
    zeros_d = jnp.zeros((_NP, _D), jnp.float32)
    t2 = node_types.reshape(_N, 1)
    m2 = (set_mask & output_mask).astype(jnp.int32).reshape(_N, 1)

    wl0 = W_state_0[:, :128, :]
    wa0 = W_state_0[:, 128:256, :]
    we0 = jnp.pad(W_state_0[:, 256:260, :], ((0, 0), (0, 124), (0, 0)))
    b0 = b_state_0.reshape(_NC, 1, _D)
    w1a = W_state_1[:, 0:128, :]
    w1b = W_state_1[:, 128:144, :]
    w1c = W_state_1[:, 144:272, :]
    wa1 = W_state_1[:, 272:400, :]
    we1 = jnp.pad(W_state_1[:, 400:404, :], ((0, 0), (0, 124), (0, 0)))
    b1 = b_state_1.reshape(_NC, 1, _D)
    wn0 = W_out_0[:128, :]
    ws0 = W_out_0[128:256, :]
    bo0 = b_out_0.reshape(1, 16)
    u1a = W_out_1[0:128, :]
    u1b = W_out_1[128:144, :]
    u1c = W_out_1[144:272, :]
    u1s = W_out_1[272:400, :]
    bo1 = b_out_1.reshape(1, 16)

    pa = _arcs_agg(arcs_p, dst_p, zeros_d)

    pre0 = _tc_call(
        _pre0_body,
        in_specs=[_row_spec(_D), _part_spec(_D), _full_spec((2, 128, _D)),
                  _full_spec((2, 128, _D)), _full_spec((2, 1, _D)),
                  _row_spec(1)],
        out_specs=[_row_spec(_D)] * 3,
        out_shape=[_f32((_N, _D))] * 3,
    )
    it_call = _tc_call(
        _iter_body,
        in_specs=[_part_spec(_D), _row_spec(_D), _row_spec(_D),
                  _full_spec((2, _D, _D)), _row_spec(1)],
        out_specs=_row_spec(_D),
        out_shape=_f32((_N, _D)),
    )
    out0_call = _tc_call(
        _out0_body,
        in_specs=[_row_spec(_D), _row_spec(_D), _full_spec((128, 16)),
                  _full_spec((128, 16)), _full_spec((1, 16)), _row_spec(1)],
        out_specs=_row_spec(16),
        out_shape=_f32((_N, 16)),
    )
    pre1 = _tc_call(
        _pre1_body,
        in_specs=[_row_spec(_D), _row_spec(16), _row_spec(_D), _part_spec(_D),
                  _full_spec((2, 128, _D)), _full_spec((2, 16, _D)),
                  _full_spec((2, 128, _D)), _full_spec((2, 128, _D)),
                  _full_spec((2, 1, _D)), _row_spec(1)],
        out_specs=[_row_spec(_D)] * 3,
        out_shape=[_f32((_N, _D))] * 3,
    )
    out1_call = _tc_call(
        _out1_body,
        in_specs=[_row_spec(_D), _row_spec(16), _row_spec(_D), _row_spec(_D),
                  _full_spec((128, 16)), _full_spec((16, 16)),
                  _full_spec((128, 16)), _full_spec((128, 16)),
                  _full_spec((1, 16))],
        out_specs=_row_spec(16),
        out_shape=_f32((_N, 16)),
    )

    l00, l01, st = pre0(nodes, pa, wl0, we0, b0, t2)
    for _ in range(2):
        p = _mp(st, src_p, dst_p, zeros_d)
        st = it_call(p, l00, l01, wa0, t2)
    out0 = out0_call(nodes, st, wn0, ws0, bo0, m2)
    s0 = st

    l10, l11, st = pre1(s0, out0, nodes, pa, w1a, w1b, w1c, we1, b1, t2)
    for _ in range(2):
        p = _mp(st, src_p, dst_p, zeros_d)
        st = it_call(p, l10, l11, wa1, t2)
    return out1_call(s0, out0, nodes, st, u1a, u1b, u1c, u1s, bo1)
